# baseline (speedup 1.0000x reference)
import functools

import jax
import jax.numpy as jnp
import numpy as np
from jax import lax
from jax.experimental import pallas as pl
from jax.experimental.pallas import tpu as pltpu
from jax.experimental.pallas import tpu_sc as plsc

INPUT_DIM = 2
LOG2_HASHMAP = 19
NUM_LEVELS = 16
F_PER_LEVEL = 2
START_RES = 16
B_SCALE = 1.447269237440378
NUM_VEC = 2 ** LOG2_HASHMAP
MASK19 = NUM_VEC - 1
PI2_I32 = np.int32(np.uint32(2654435761).view(np.int32))
RES = [int(B_SCALE ** i * START_RES) for i in range(NUM_LEVELS)]

NC = 2
NS = 16
NW = NC * NS

B = 262144
PTS_PER_W = B // NW
C = 1024
NCHUNK = PTS_PER_W // C
NGRP = C // 16
NROW = (C * 4) // 128


def _encode_body(x_hbm, tab_hbm, out_hbm,
                 idx_a, idx_b, rows_a, rows_b, x_v, out_v,
                 sem_a, sem_b):
    iota = lax.iota(jnp.int32, 16)
    zero16 = jnp.zeros((16,), jnp.int32)
    one16 = jnp.ones((16,), jnp.int32)
    idx_refs = (idx_a, idx_b)
    rows_refs = (rows_a, rows_b)
    sems = (sem_a, sem_b)

    wid = lax.axis_index("s") * NC + lax.axis_index("c")

    def frac_coords(g, res_f):
        pr = g * 16 + iota
        p0 = plsc.load_gather(x_v, [pr, zero16])
        p1 = plsc.load_gather(x_v, [pr, one16])
        xr0 = p0 * res_f
        xr1 = p1 * res_f
        i0 = xr0.astype(jnp.int32)
        i1 = xr1.astype(jnp.int32)
        d0 = xr0 - i0.astype(jnp.float32)
        d1 = xr1 - i1.astype(jnp.float32)
        return i0, i1, d0, d1

    def gen_fire(l, sel):
        res_f = np.float32(RES[l])
        idx_ref = idx_refs[sel]
        rows_ref = rows_refs[sel]
        sem = sems[sel]

        def body(t, carry):
            for k in range(2):
                g = t * 2 + k
                i0, i1, _, _ = frac_coords(g, res_f)
                b0 = i1 * PI2_I32
                b1 = b0 + PI2_I32
                a1 = i0 + 1
                h00 = (i0 ^ b0) & MASK19
                h01 = (i0 ^ b1) & MASK19
                h10 = (a1 ^ b0) & MASK19
                h11 = (a1 ^ b1) & MASK19
                s = k * 4
                idx_ref[t, pl.ds((s + 0) * 16, 16)] = h00
                idx_ref[t, pl.ds((s + 1) * 16, 16)] = h01
                idx_ref[t, pl.ds((s + 2) * 16, 16)] = h10
                idx_ref[t, pl.ds((s + 3) * 16, 16)] = h11
            pltpu.make_async_copy(
                tab_hbm.at[l].at[idx_ref.at[t]],
                rows_ref.at[pl.ds(t * 128, 128), :],
                sem,
            ).start()
            return carry

        lax.fori_loop(0, NROW, body, 0)

    def drain(l, sel):
        idx_ref = idx_refs[sel]
        rows_ref = rows_refs[sel]
        sem = sems[sel]

        def body(t, carry):
            pltpu.make_async_copy(
                tab_hbm.at[l].at[idx_ref.at[t]],
                rows_ref.at[pl.ds(t * 128, 128), :],
                sem,
            ).wait()
            return carry

        lax.fori_loop(0, NROW, body, 0)

    def interp(l, sel):
        res_f = np.float32(RES[l])
        rows_ref = rows_refs[sel]
        col0 = jnp.full((16,), 2 * l, jnp.int32)
        col1 = jnp.full((16,), 2 * l + 1, jnp.int32)

        def body(t, carry):
            for k in range(2):
                g = t * 2 + k
                _, _, d0, d1 = frac_coords(g, res_f)
                r0 = g * 64 + iota
                v = []
                for c in range(4):
                    rc = r0 + (c * 16)
                    v.append((plsc.load_gather(rows_ref, [rc, zero16]),
                              plsc.load_gather(rows_ref, [rc, one16])))
                pr = g * 16 + iota
                for f, col in ((0, col0), (1, col1)):
                    c0 = v[0][f] + d0 * (v[2][f] - v[0][f])
                    c1 = v[1][f] + d0 * (v[3][f] - v[1][f])
                    cf = c0 + d1 * (c1 - c0)
                    plsc.store_scatter(out_v, [pr, col], cf)
            return carry

        lax.fori_loop(0, NROW, body, 0)

    def chunk_body(n, carry):
        base = (wid * NCHUNK + n) * C
        pltpu.sync_copy(x_hbm.at[pl.ds(base, C)], x_v)
        gen_fire(0, 0)
        for l in range(NUM_LEVELS):
            sel = l & 1
            if l + 1 < NUM_LEVELS:
                gen_fire(l + 1, 1 - sel)
            drain(l, sel)
            interp(l, sel)
        pltpu.sync_copy(out_v, out_hbm.at[pl.ds(base, C)])
        return carry

    lax.fori_loop(0, NCHUNK, chunk_body, 0)


@functools.partial(
    pl.kernel,
    out_type=jax.ShapeDtypeStruct((B, NUM_LEVELS * F_PER_LEVEL), jnp.float32),
    mesh=plsc.VectorSubcoreMesh(
        core_axis_name="c", subcore_axis_name="s",
        num_cores=NC, num_subcores=NS),
    compiler_params=pltpu.CompilerParams(
        needs_layout_passes=False, use_tc_tiling_on_sc=False),
    scratch_types=[
        pltpu.VMEM((NROW, 128), jnp.int32),
        pltpu.VMEM((NROW, 128), jnp.int32),
        pltpu.VMEM((C * 4, 2), jnp.float32),
        pltpu.VMEM((C * 4, 2), jnp.float32),
        pltpu.VMEM((C, INPUT_DIM), jnp.float32),
        pltpu.VMEM((C, NUM_LEVELS * F_PER_LEVEL), jnp.float32),
        pltpu.SemaphoreType.DMA,
        pltpu.SemaphoreType.DMA,
    ],
)
def _encode(*refs):
    _encode_body(*refs)


def kernel(x, tables):
    return _encode(x, tables)


# speedup vs baseline: 7.7711x; 7.7711x over previous
import functools

import jax
import jax.numpy as jnp
import numpy as np
from jax import lax
from jax.experimental import pallas as pl
from jax.experimental.pallas import tpu as pltpu
from jax.experimental.pallas import tpu_sc as plsc

INPUT_DIM = 2
LOG2_HASHMAP = 19
NUM_LEVELS = 16
F_PER_LEVEL = 2
START_RES = 16
B_SCALE = 1.447269237440378
NUM_VEC = 2 ** LOG2_HASHMAP
MASK19 = NUM_VEC - 1
PI2_I32 = np.int32(np.uint32(2654435761).view(np.int32))
RES = [int(B_SCALE ** i * START_RES) for i in range(NUM_LEVELS)]

NC = 2
NS = 16
NW = NC * NS

B = 262144
NF = NUM_LEVELS * F_PER_LEVEL
PTS_PER_W = B // NW
C = 1024
NCHUNK = PTS_PER_W // C
NGRP = C // 16
PTILE = C // 128


def _encode_body(x_hbm, tab_hbm, out_hbm,
                 idx_a, idx_b, rows_a, rows_b, x_v, out_v,
                 sem_a, sem_b):
    idx_refs = (idx_a, idx_b)
    rows_refs = (rows_a, rows_b)
    sems = (sem_a, sem_b)

    wid = lax.axis_index("s") * NC + lax.axis_index("c")

    def frac_coords(g, res_f):
        q = lax.div(g, 8)
        s = lax.rem(g, 8) * 16
        p0 = x_v[q, 0, pl.ds(s, 16)]
        p1 = x_v[q, 1, pl.ds(s, 16)]
        xr0 = p0 * res_f
        xr1 = p1 * res_f
        i0 = xr0.astype(jnp.int32)
        i1 = xr1.astype(jnp.int32)
        d0 = xr0 - i0.astype(jnp.float32)
        d1 = xr1 - i1.astype(jnp.float32)
        return i0, i1, d0, d1

    def flat_pair(v, loff):
        vf0 = (v + (v & -128)) + loff
        return vf0, vf0 + 128

    def gen_fire(l, sel):
        res_f = np.float32(RES[l])
        loff = np.int32(l << (LOG2_HASHMAP + 1))
        idx_ref = idx_refs[sel]
        rows_ref = rows_refs[sel]
        sem = sems[sel]

        def body(g, carry):
            i0, i1, _, _ = frac_coords(g, res_f)
            b0 = i1 * PI2_I32
            b1 = b0 + PI2_I32
            a1 = i0 + 1
            h00 = (i0 ^ b0) & MASK19
            h01 = (i0 ^ b1) & MASK19
            h10 = (a1 ^ b0) & MASK19
            h11 = (a1 ^ b1) & MASK19
            for c, h in enumerate((h00, h01, h10, h11)):
                f0, f1 = flat_pair(h, loff)
                idx_ref[g, pl.ds(c * 32, 16)] = f0
                idx_ref[g, pl.ds(c * 32 + 16, 16)] = f1
            pltpu.make_async_copy(
                tab_hbm.at[idx_ref.at[g]],
                rows_ref.at[pl.ds(g * 128, 128)],
                sem,
            ).start()
            return carry

        lax.fori_loop(0, NGRP, body, 0)

    def drain(sel):
        idx_ref = idx_refs[sel]
        rows_ref = rows_refs[sel]
        sem = sems[sel]

        def body(g, carry):
            pltpu.make_async_copy(
                tab_hbm.at[idx_ref.at[g]],
                rows_ref.at[pl.ds(g * 128, 128)],
                sem,
            ).wait()
            return carry

        lax.fori_loop(0, NGRP, body, 0)

    def interp(l, sel):
        res_f = np.float32(RES[l])
        rows_ref = rows_refs[sel]
        tr0, s0 = (2 * l) // 8, (2 * l) % 8
        tr1, s1 = (2 * l + 1) // 8, (2 * l + 1) % 8

        def body(g, carry):
            _, _, d0, d1 = frac_coords(g, res_f)
            rbase = g * 128
            v = []
            for c in range(4):
                v.append((rows_ref[pl.ds(rbase + c * 32, 16)],
                          rows_ref[pl.ds(rbase + c * 32 + 16, 16)]))
            qc = lax.div(g, 8)
            cb = lax.rem(g, 8) * 16
            for f, (tr, s) in ((0, (tr0, s0)), (1, (tr1, s1))):
                c0 = v[0][f] + d0 * (v[2][f] - v[0][f])
                c1 = v[1][f] + d0 * (v[3][f] - v[1][f])
                cf = c0 + d1 * (c1 - c0)
                out_v[tr, qc, s, pl.ds(cb, 16)] = cf
            return carry

        lax.fori_loop(0, NGRP, body, 0)

    def chunk_body(n, carry):
        t0 = (wid * NCHUNK + n) * PTILE
        pltpu.sync_copy(x_hbm.at[pl.ds(t0, PTILE)], x_v)
        gen_fire(0, 0)
        for l in range(NUM_LEVELS):
            sel = l & 1
            if l + 1 < NUM_LEVELS:
                gen_fire(l + 1, 1 - sel)
            drain(sel)
            interp(l, sel)
        for tr in range(NF // 8):
            pltpu.sync_copy(out_v.at[tr], out_hbm.at[tr, pl.ds(t0, PTILE)])
        return carry

    lax.fori_loop(0, NCHUNK, chunk_body, 0)


@functools.partial(
    pl.kernel,
    out_type=jax.ShapeDtypeStruct((NF // 8, B // 128, 8, 128), jnp.float32),
    mesh=plsc.VectorSubcoreMesh(
        core_axis_name="c", subcore_axis_name="s",
        num_cores=NC, num_subcores=NS),
    compiler_params=pltpu.CompilerParams(
        needs_layout_passes=False, use_tc_tiling_on_sc=False),
    scratch_types=[
        pltpu.VMEM((NGRP, 128), jnp.int32),
        pltpu.VMEM((NGRP, 128), jnp.int32),
        pltpu.VMEM((C * 8,), jnp.float32),
        pltpu.VMEM((C * 8,), jnp.float32),
        pltpu.VMEM((PTILE, INPUT_DIM, 128), jnp.float32),
        pltpu.VMEM((NF // 8, PTILE, 8, 128), jnp.float32),
        pltpu.SemaphoreType.DMA,
        pltpu.SemaphoreType.DMA,
    ],
)
def _encode(*refs):
    _encode_body(*refs)


def kernel(x, tables):
    xv = x.reshape(B // 128, 128, INPUT_DIM).transpose(0, 2, 1)
    tabv = (tables.reshape(NUM_LEVELS, NUM_VEC // 128, 128, F_PER_LEVEL)
            .transpose(0, 1, 3, 2)
            .reshape(NUM_LEVELS * NUM_VEC * F_PER_LEVEL))
    out4 = _encode(xv, tabv)
    return out4.transpose(1, 3, 0, 2).reshape(B, NF)


# speedup vs baseline: 29.5565x; 3.8034x over previous
import functools

import jax
import jax.numpy as jnp
import numpy as np
from jax import lax
from jax.experimental import pallas as pl
from jax.experimental.pallas import tpu as pltpu
from jax.experimental.pallas import tpu_sc as plsc

INPUT_DIM = 2
LOG2_HASHMAP = 19
NUM_LEVELS = 16
F_PER_LEVEL = 2
START_RES = 16
B_SCALE = 1.447269237440378
NUM_VEC = 2 ** LOG2_HASHMAP
MASK19 = NUM_VEC - 1
PI2_I32 = np.int32(np.uint32(2654435761).view(np.int32))
RES = [int(B_SCALE ** i * START_RES) for i in range(NUM_LEVELS)]

NC = 2
NS = 16
NW = NC * NS

B = 262144
NF = NUM_LEVELS * F_PER_LEVEL
PTS_PER_W = B // NW
C = 1024
NCHUNK = PTS_PER_W // C
NGRP = C // 16
PTILE = C // 128
LVL_F32 = NUM_VEC * F_PER_LEVEL
STAGE_F32 = LVL_F32 // NS


def _encode_body(x_hbm, tab_hbm, out_hbm,
                 tabs_s, idx_v, rows_v, x_v, obuf, sem):
    wid = lax.axis_index("s") * NC + lax.axis_index("c")
    sid = lax.axis_index("s")

    def frac_coords(n, g, res_f):
        q = n * PTILE + lax.div(g, 8)
        s = lax.rem(g, 8) * 16
        p0 = x_v[q, 0, pl.ds(s, 16)]
        p1 = x_v[q, 1, pl.ds(s, 16)]
        xr0 = p0 * res_f
        xr1 = p1 * res_f
        i0 = xr0.astype(jnp.int32)
        i1 = xr1.astype(jnp.int32)
        d0 = xr0 - i0.astype(jnp.float32)
        d1 = xr1 - i1.astype(jnp.float32)
        return i0, i1, d0, d1

    def gen_fire(l, n, sel):
        res_f = np.float32(RES[l])

        def body(g, carry):
            i0, i1, _, _ = frac_coords(n, g, res_f)
            b0 = i1 * PI2_I32
            b1 = b0 + PI2_I32
            a1 = i0 + 1
            h00 = (i0 ^ b0) & MASK19
            h01 = (i0 ^ b1) & MASK19
            h10 = (a1 ^ b0) & MASK19
            h11 = (a1 ^ b1) & MASK19
            for c, h in enumerate((h00, h01, h10, h11)):
                f0 = h + (h & -128)
                idx_v[sel, g, pl.ds(c * 32, 16)] = f0
                idx_v[sel, g, pl.ds(c * 32 + 16, 16)] = f0 + 128
            pltpu.make_async_copy(
                tabs_s.at[idx_v.at[sel, g]],
                rows_v.at[sel, pl.ds(g * 128, 128)],
                sem.at[sel],
            ).start()
            return carry

        lax.fori_loop(0, NGRP, body, 0)

    def drain(sel):
        def body(g, carry):
            pltpu.make_async_copy(
                tabs_s.at[idx_v.at[sel, g]],
                rows_v.at[sel, pl.ds(g * 128, 128)],
                sem.at[sel],
            ).wait()
            return carry

        lax.fori_loop(0, NGRP, body, 0)

    def interp(l, n, sel):
        res_f = np.float32(RES[l])

        def body(g, carry):
            _, _, d0, d1 = frac_coords(n, g, res_f)
            rbase = g * 128
            v = []
            for c in range(4):
                v.append((rows_v[sel, pl.ds(rbase + c * 32, 16)],
                          rows_v[sel, pl.ds(rbase + c * 32 + 16, 16)]))
            qc = lax.div(g, 8)
            cb = lax.rem(g, 8) * 16
            for f in range(2):
                c0 = v[0][f] + d0 * (v[2][f] - v[0][f])
                c1 = v[1][f] + d0 * (v[3][f] - v[1][f])
                cf = c0 + d1 * (c1 - c0)
                obuf[f, qc, 0, pl.ds(cb, 16)] = cf
            return carry

        lax.fori_loop(0, NGRP, body, 0)

    pltpu.sync_copy(x_hbm.at[pl.ds(wid * (PTS_PER_W // 128), PTS_PER_W // 128)],
                    x_v)

    for l in range(NUM_LEVELS):
        pltpu.sync_copy(
            tab_hbm.at[pl.ds(l * LVL_F32 + sid * STAGE_F32, STAGE_F32)],
            tabs_s.at[pl.ds(sid * STAGE_F32, STAGE_F32)])
        plsc.subcore_barrier()

        gen_fire(l, 0, 0)

        def chunk_body(n, carry, l=l):
            sel = lax.rem(n, 2)
            nsel = 1 - sel

            @pl.when(n + 1 < NCHUNK)
            def _():
                gen_fire(l, n + 1, nsel)

            drain(sel)
            interp(l, n, sel)
            t0 = wid * (PTS_PER_W // 128) + n * PTILE
            for f in range(2):
                ff = 2 * l + f
                pltpu.sync_copy(
                    obuf.at[f],
                    out_hbm.at[ff // 8, pl.ds(t0, PTILE), pl.ds(ff % 8, 1)])
            return carry

        lax.fori_loop(0, NCHUNK, chunk_body, 0)
        plsc.subcore_barrier()


@functools.partial(
    pl.kernel,
    out_type=jax.ShapeDtypeStruct((NF // 8, B // 128, 8, 128), jnp.float32),
    mesh=plsc.VectorSubcoreMesh(
        core_axis_name="c", subcore_axis_name="s",
        num_cores=NC, num_subcores=NS),
    compiler_params=pltpu.CompilerParams(
        needs_layout_passes=False, use_tc_tiling_on_sc=False),
    scratch_types=[
        pltpu.VMEM_SHARED((LVL_F32,), jnp.float32),
        pltpu.VMEM((2, NGRP, 128), jnp.int32),
        pltpu.VMEM((2, C * 8), jnp.float32),
        pltpu.VMEM((PTS_PER_W // 128, INPUT_DIM, 128), jnp.float32),
        pltpu.VMEM((F_PER_LEVEL, PTILE, 1, 128), jnp.float32),
        pltpu.SemaphoreType.DMA((2,)),
    ],
)
def _encode(*refs):
    _encode_body(*refs)


def kernel(x, tables):
    xv = x.reshape(B // 128, 128, INPUT_DIM).transpose(0, 2, 1)
    tabv = (tables.reshape(NUM_LEVELS, NUM_VEC // 128, 128, F_PER_LEVEL)
            .transpose(0, 1, 3, 2)
            .reshape(NUM_LEVELS * NUM_VEC * F_PER_LEVEL))
    out4 = _encode(xv, tabv)
    return out4.transpose(1, 3, 0, 2).reshape(B, NF)
